# baseline (speedup 1.0000x reference)
import numpy as np
import jax
import jax.numpy as jnp
from jax.experimental import pallas as pl
from jax.experimental.pallas import tpu as pltpu

_EPS = 1e-5
_B = 8192
_L = 2048
_FOLD = 128
_NCH = _L // _FOLD
_NPART = 15
_SB1 = 128
_SB2 = 128
_STD = float(np.sqrt((1.0 + _L * _EPS) / _L))
_SEM = "arbitrary"
_NCORE = 2


def _build_w() -> np.ndarray:
    w = np.zeros((_NPART * _FOLD, 16), np.float32)
    for q in range(_NPART):
        w[q * _FOLD:(q + 1) * _FOLD, q] = 1.0
    return w


_W_NP = _build_w()


def _stats_body(x_ref, p_ref, w_ref, g_ref, o_ref, ot_ref, onb_ref):
    xs = [x_ref[c] for c in range(3)]
    ps = [p_ref[c] for c in range(3)]

    def fold(v):
        acc = v[:, :_FOLD]
        for i in range(1, _NCH):
            acc = acc + v[:, i * _FOLD:(i + 1) * _FOLD]
        return acc

    parts = [fold(v) for v in xs] + [fold(v) for v in ps]
    for k in range(3):
        for j in range(3):
            parts.append(fold(ps[k] * xs[j]))
    v = jnp.concatenate(parts, axis=1)
    o_ref[...] = jax.lax.dot_general(
        v, w_ref[...], (((1,), (0,)), ((), ())),
        preferred_element_type=jnp.float32)
    ot_ref[...] = jax.lax.dot_general(
        w_ref[...], v, (((0,), (1,)), ((), ())),
        preferred_element_type=jnp.float32)
    inv_l = np.float32(1.0 / _L)
    gg = jnp.concatenate([g_ref[...]] * (_SB1 // 8), axis=0)
    for c in range(3):
        mu_c = jnp.sum(parts[c], axis=1, keepdims=True) * inv_l
        onb_ref[c] = gg * (xs[c] - mu_c)


def _gsp_body(s_ref, b_ref, o_ref):
    inv_l = np.float32(1.0 / _L)
    mu = [s_ref[c] * inv_l for c in range(3)]
    sp = [s_ref[3 + c] for c in range(3)]
    m = [[s_ref[6 + 3 * k + j] for j in range(3)] for k in range(3)]
    init = [[m[k][j] - sp[k] * mu[j] for j in range(3)] for k in range(3)]

    def dot3(a, b2):
        return a[0] * b2[0] + a[1] * b2[1] + a[2] * b2[2]

    v = []
    for k in range(3):
        nrm = jnp.sqrt(dot3(init[k], init[k]))
        v.append([init[k][j] / nrm for j in range(3)])
    u0 = v[0]
    d00 = dot3(u0, u0) + _EPS
    c10 = dot3(v[1], u0) / d00
    u1 = [v[1][j] - c10 * u0[j] for j in range(3)]
    c20 = dot3(v[2], u0) / d00
    c21 = dot3(v[2], u1) / (dot3(u1, u1) + _EPS)
    u2 = [v[2][j] - c20 * u0[j] - c21 * u1[j] for j in range(3)]
    un = []
    for uk in (u0, u1, u2):
        nrm = jnp.sqrt(dot3(uk, uk)) + _EPS
        un.append([uk[j] / nrm for j in range(3)])
    for i in range(3):
        o_ref[i] = (un[0][i] * b_ref[0] + un[1][i] * b_ref[1]
                    + un[2][i] * b_ref[2])


def _apply_body(x_ref, s_ref, g_ref, b_ref, o_ref):
    reps = _SB2 // 8
    inv_l = np.float32(1.0 / _L)
    gg = jnp.concatenate([g_ref[...]] * reps, axis=0)
    for c in range(3):
        mu_c = s_ref[:, c:c + 1] * inv_l
        bb = jnp.concatenate([b_ref[c]] * reps, axis=0)
        o_ref[c] = gg * (x_ref[c] - mu_c) + bb


def kernel(x, g, b, proj):
    xt = x.T.reshape(3, _B, _L)
    pt = proj.T.reshape(3, _B, _L)
    w = jnp.asarray(_W_NP)

    g8 = jnp.tile((g * np.float32(1.0 / _STD)).reshape(4, _L), (2, 1))
    ns1 = _B // _SB1 // _NCORE
    stats, stats_tr, out_nb = pl.pallas_call(
        _stats_body,
        out_shape=(jax.ShapeDtypeStruct((_B, 16), jnp.float32),
                   jax.ShapeDtypeStruct((16, _B), jnp.float32),
                   jax.ShapeDtypeStruct((3, _B, _L), jnp.float32)),
        grid=(_NCORE, ns1),
        in_specs=[
            pl.BlockSpec((3, _SB1, _L), lambda c, i: (0, c * ns1 + i, 0)),
            pl.BlockSpec((3, _SB1, _L), lambda c, i: (0, c * ns1 + i, 0)),
            pl.BlockSpec((_NPART * _FOLD, 16), lambda c, i: (0, 0)),
            pl.BlockSpec((8, _L), lambda c, i: (0, 0)),
        ],
        out_specs=(pl.BlockSpec((_SB1, 16), lambda c, i: (c * ns1 + i, 0)),
                   pl.BlockSpec((16, _SB1), lambda c, i: (0, c * ns1 + i)),
                   pl.BlockSpec((3, _SB1, _L),
                                lambda c, i: (0, c * ns1 + i, 0))),
        compiler_params=pltpu.CompilerParams(
            dimension_semantics=(_SEM, "arbitrary"),
            vmem_limit_bytes=48 * 1024 * 1024,
        ),
        name="eqnorm_stats",
    )(xt, pt, w, g8)

    stats_t = stats_tr.reshape(16, 64, 128)
    b_t = b.T.reshape(3, 64, 128)
    bias_t = pl.pallas_call(
        _gsp_body,
        out_shape=jax.ShapeDtypeStruct((3, 64, 128), jnp.float32),
        grid=(2,),
        in_specs=[
            pl.BlockSpec((16, 32, 128), lambda i: (0, i, 0)),
            pl.BlockSpec((3, 32, 128), lambda i: (0, i, 0)),
        ],
        out_specs=pl.BlockSpec((3, 32, 128), lambda i: (0, i, 0)),
        compiler_params=pltpu.CompilerParams(
            dimension_semantics=(_SEM,),
        ),
        name="eqnorm_gsp",
    )(stats_t, b_t)

    bias_i = bias_t.reshape(3, _B).T
    out = out_nb.reshape(3, _B * _L).T + jnp.tile(bias_i, (_L, 1))
    return out


# speedup vs baseline: 1.5629x; 1.5629x over previous
import numpy as np
import jax
import jax.numpy as jnp
from jax.experimental import pallas as pl
from jax.experimental.pallas import tpu as pltpu

_EPS = 1e-5
_B = 8192
_L = 2048
_FOLD = 128
_NCH = _L // _FOLD
_NPART = 15
_SB1 = 256
_SB2 = 256
_STD = float(np.sqrt((1.0 + _L * _EPS) / _L))
_SEM = "arbitrary"
_NCORE = 2


def _build_w() -> np.ndarray:
    w = np.zeros((_NPART * _FOLD, 16), np.float32)
    for q in range(_NPART):
        w[q * _FOLD:(q + 1) * _FOLD, q] = 1.0
    return w


_W_NP = _build_w()


def _stats_body(x_ref, p_ref, w_ref, o_ref, ot_ref):
    xs = [x_ref[c] for c in range(3)]
    ps = [p_ref[c] for c in range(3)]

    def fold(v):
        acc = v[:, :_FOLD]
        for i in range(1, _NCH):
            acc = acc + v[:, i * _FOLD:(i + 1) * _FOLD]
        return acc

    parts = [fold(v) for v in xs] + [fold(v) for v in ps]
    for k in range(3):
        for j in range(3):
            parts.append(fold(ps[k] * xs[j]))
    v = jnp.concatenate(parts, axis=1)
    o_ref[...] = jax.lax.dot_general(
        v, w_ref[...], (((1,), (0,)), ((), ())),
        preferred_element_type=jnp.float32)
    ot_ref[...] = jax.lax.dot_general(
        w_ref[...], v, (((0,), (1,)), ((), ())),
        preferred_element_type=jnp.float32)


def _gsp_body(s_ref, b_ref, o_ref):
    inv_l = np.float32(1.0 / _L)
    mu = [s_ref[c] * inv_l for c in range(3)]
    sp = [s_ref[3 + c] for c in range(3)]
    m = [[s_ref[6 + 3 * k + j] for j in range(3)] for k in range(3)]
    init = [[m[k][j] - sp[k] * mu[j] for j in range(3)] for k in range(3)]

    def dot3(a, b2):
        return a[0] * b2[0] + a[1] * b2[1] + a[2] * b2[2]

    v = []
    for k in range(3):
        nrm = jnp.sqrt(dot3(init[k], init[k]))
        v.append([init[k][j] / nrm for j in range(3)])
    u0 = v[0]
    d00 = dot3(u0, u0) + _EPS
    c10 = dot3(v[1], u0) / d00
    u1 = [v[1][j] - c10 * u0[j] for j in range(3)]
    c20 = dot3(v[2], u0) / d00
    c21 = dot3(v[2], u1) / (dot3(u1, u1) + _EPS)
    u2 = [v[2][j] - c20 * u0[j] - c21 * u1[j] for j in range(3)]
    un = []
    for uk in (u0, u1, u2):
        nrm = jnp.sqrt(dot3(uk, uk)) + _EPS
        un.append([uk[j] / nrm for j in range(3)])
    for i in range(3):
        o_ref[i] = (un[0][i] * b_ref[0] + un[1][i] * b_ref[1]
                    + un[2][i] * b_ref[2])


def _apply_body(x_ref, s_ref, g_ref, b_ref, o_ref):
    reps = _SB2 // 8
    inv_l = np.float32(1.0 / _L)
    gg = jnp.concatenate([g_ref[...]] * reps, axis=0)
    for c in range(3):
        mu_c = s_ref[:, c:c + 1] * inv_l
        bb = jnp.concatenate([b_ref[c]] * reps, axis=0)
        o_ref[c] = gg * (x_ref[c] - mu_c) + bb


def kernel(x, g, b, proj):
    xt = x.T.reshape(3, _B, _L)
    pt = proj.T.reshape(3, _B, _L)
    w = jnp.asarray(_W_NP)

    ns1 = _B // _SB1 // _NCORE
    stats, stats_tr = pl.pallas_call(
        _stats_body,
        out_shape=(jax.ShapeDtypeStruct((_B, 16), jnp.float32),
                   jax.ShapeDtypeStruct((16, _B), jnp.float32)),
        grid=(_NCORE, ns1),
        in_specs=[
            pl.BlockSpec((3, _SB1, _L), lambda c, i: (0, c * ns1 + i, 0)),
            pl.BlockSpec((3, _SB1, _L), lambda c, i: (0, c * ns1 + i, 0)),
            pl.BlockSpec((_NPART * _FOLD, 16), lambda c, i: (0, 0)),
        ],
        out_specs=(pl.BlockSpec((_SB1, 16), lambda c, i: (c * ns1 + i, 0)),
                   pl.BlockSpec((16, _SB1), lambda c, i: (0, c * ns1 + i))),
        compiler_params=pltpu.CompilerParams(
            dimension_semantics=(_SEM, "arbitrary"),
            vmem_limit_bytes=48 * 1024 * 1024,
        ),
        name="eqnorm_stats",
    )(xt, pt, w)

    stats_t = stats_tr.reshape(16, 64, 128)
    b_t = b.T.reshape(3, 64, 128)
    bias_t = pl.pallas_call(
        _gsp_body,
        out_shape=jax.ShapeDtypeStruct((3, 64, 128), jnp.float32),
        grid=(2,),
        in_specs=[
            pl.BlockSpec((16, 32, 128), lambda i: (0, i, 0)),
            pl.BlockSpec((3, 32, 128), lambda i: (0, i, 0)),
        ],
        out_specs=pl.BlockSpec((3, 32, 128), lambda i: (0, i, 0)),
        compiler_params=pltpu.CompilerParams(
            dimension_semantics=(_SEM,),
        ),
        name="eqnorm_gsp",
    )(stats_t, b_t)

    b8 = jnp.tile(bias_t.reshape(3, 4, _L), (1, 2, 1))
    g8 = jnp.tile((g * np.float32(1.0 / _STD)).reshape(4, _L), (2, 1))

    ns2 = _B // _SB2 // _NCORE
    out_t = pl.pallas_call(
        _apply_body,
        out_shape=jax.ShapeDtypeStruct((3, _B, _L), jnp.float32),
        grid=(_NCORE, ns2),
        in_specs=[
            pl.BlockSpec((3, _SB2, _L), lambda c, i: (0, c * ns2 + i, 0)),
            pl.BlockSpec((_SB2, 16), lambda c, i: (c * ns2 + i, 0)),
            pl.BlockSpec((8, _L), lambda c, i: (0, 0)),
            pl.BlockSpec((3, 8, _L), lambda c, i: (0, 0, 0)),
        ],
        out_specs=pl.BlockSpec((3, _SB2, _L), lambda c, i: (0, c * ns2 + i, 0)),
        compiler_params=pltpu.CompilerParams(
            dimension_semantics=(_SEM, "arbitrary"),
            vmem_limit_bytes=48 * 1024 * 1024,
        ),
        name="eqnorm_apply",
    )(xt, stats, g8, b8)

    return out_t.reshape(3, _B * _L).T
